# baseline (speedup 1.0000x reference)
import jax
import jax.numpy as jnp
from jax.experimental import pallas as pl
from jax.experimental.pallas import tpu as pltpu
from jax.experimental.pallas import tpu_sc as plsc

BATCH = 256
CW_DIM = 2048
D = 256
K = 8192
C = CW_DIM // D

GATHER_WINDOW = 128

KCHUNK = 512
NKB = K // KCHUNK
LANES = 128
NSLAB = KCHUNK // LANES
INT_BIG = 2**31 - 1


def _argmin_onehot_kernel(x_ref, e_ref, oh_ref, idx_ref, rm_ref, rk_ref, amin_ref, sc_ref):
    c = pl.program_id(0)
    p = pl.program_id(1)
    kb = pl.program_id(2)

    @pl.when(p == 0)
    def _score_phase():
        x = x_ref[...]
        e = e_ref[0]
        xe = jax.lax.dot_general(
            x, e, (((1,), (1,)), ((), ())),
            preferred_element_type=jnp.float32,
            precision=jax.lax.Precision.DEFAULT,
        )
        e2row = jax.lax.dot_general(
            jnp.ones((1, D), jnp.float32), e * e, (((1,), (1,)), ((), ())),
            preferred_element_type=jnp.float32,
            precision=jax.lax.Precision.HIGHEST,
        )
        x2col = jax.lax.dot_general(
            x * x, jnp.ones((1, D), jnp.float32), (((1,), (1,)), ((), ())),
            preferred_element_type=jnp.float32,
            precision=jax.lax.Precision.HIGHEST,
        )
        sc_ref[...] = (x2col - 2.0 * xe) + e2row
        scores = sc_ref[...]

        @pl.when(kb == 0)
        def _():
            rm_ref[...] = jnp.full((BATCH, LANES), jnp.inf, jnp.float32)
            rk_ref[...] = jnp.zeros((BATCH, LANES), jnp.int32)

        rm = rm_ref[...]
        rk = rk_ref[...]
        lane_iota = jax.lax.broadcasted_iota(jnp.int32, (BATCH, LANES), 1)
        for s in range(NSLAB):
            slab = scores[:, s * LANES:(s + 1) * LANES]
            kvec = lane_iota + (kb * KCHUNK + s * LANES)
            cond = slab < rm
            rm = jnp.where(cond, slab, rm)
            rk = jnp.where(cond, kvec, rk)
        rm_ref[...] = rm
        rk_ref[...] = rk

        @pl.when(kb == NKB - 1)
        def _():
            lm = jnp.min(rm, axis=1, keepdims=True)
            cand = jnp.where(rm == lm, rk, INT_BIG)
            amin_ref[...] = jnp.min(cand, axis=1, keepdims=True)

    @pl.when(p == 1)
    def _onehot_phase():
        idx = amin_ref[...]
        k_iota = jax.lax.broadcasted_iota(jnp.int32, (BATCH, KCHUNK), 1) + kb * KCHUNK
        oh_ref[...] = (k_iota == idx).astype(jnp.float32)

        @pl.when(kb == NKB - 1)
        def _():
            idx_ref[0] = idx + c * K


def _sc_gather(table_flat, idx_flat):
    n_idx = BATCH * C
    mesh = plsc.VectorSubcoreMesh(core_axis_name="core", subcore_axis_name="subcore")

    @pl.kernel(out_type=jax.ShapeDtypeStruct((n_idx, D), jnp.float32), mesh=mesh)
    def gather_kernel(tab_hbm, i_hbm, o_hbm):
        def body(i_vmem, o_vmem):
            pltpu.sync_copy(tab_hbm.at[i_vmem.at[0]], o_vmem)

        pltpu.emit_pipeline(
            body,
            grid=(n_idx // GATHER_WINDOW,),
            in_specs=[pl.BlockSpec((1, GATHER_WINDOW), index_map=lambda i: (0, i))],
            out_specs=[pl.BlockSpec((GATHER_WINDOW, D), index_map=lambda i: (i, 0))],
            core_axis_name=("core", "subcore"),
            dimension_semantics=(pltpu.PARALLEL,),
        )(i_hbm, o_hbm)

    return gather_kernel(table_flat, idx_flat)


def kernel(cw_q, codebook):
    one_hot_flat, idx_out = pl.pallas_call(
        _argmin_onehot_kernel,
        grid=(C, 2, NKB),
        in_specs=[
            pl.BlockSpec((BATCH, D), lambda c, p, kb: (0, c)),
            pl.BlockSpec((1, KCHUNK, D),
                         lambda c, p, kb: (c, kb + p * (NKB - 1 - kb), 0)),
        ],
        out_specs=[
            pl.BlockSpec((BATCH, KCHUNK),
                         lambda c, p, kb: (0, c * NKB + p * kb)),
            pl.BlockSpec((1, BATCH, 1), lambda c, p, kb: (c, 0, 0)),
        ],
        out_shape=[
            jax.ShapeDtypeStruct((BATCH, C * K), jnp.float32),
            jax.ShapeDtypeStruct((C, BATCH, 1), jnp.int32),
        ],
        scratch_shapes=[
            pltpu.VMEM((BATCH, LANES), jnp.float32),
            pltpu.VMEM((BATCH, LANES), jnp.int32),
            pltpu.VMEM((BATCH, 1), jnp.int32),
            pltpu.VMEM((BATCH, KCHUNK), jnp.float32),
        ],
    )(cw_q, codebook)

    one_hot = one_hot_flat.reshape(BATCH, C, K)
    idx_flat = idx_out[:, :, 0].T.reshape(1, BATCH * C)
    closest = _sc_gather(codebook.reshape(C * K, D), idx_flat)
    cw_e = closest.reshape(BATCH, CW_DIM)
    cw = cw_q + jax.lax.stop_gradient(cw_e - cw_q)
    return cw, one_hot


# speedup vs baseline: 1.5135x; 1.5135x over previous
import jax
import jax.numpy as jnp
from jax.experimental import pallas as pl
from jax.experimental.pallas import tpu as pltpu
from jax.experimental.pallas import tpu_sc as plsc

BATCH = 256
CW_DIM = 2048
D = 256
K = 8192
C = CW_DIM // D

GATHER_WINDOW = 128

KCHUNK = 512
NKB = K // KCHUNK
LANES = 128
NSLAB = KCHUNK // LANES
INT_BIG = 2**31 - 1


def _argmin_kernel(x_ref, e_ref, idxsc_ref, idxbc_ref,
                   rm_ref, rk_ref, aall_ref, sc_ref):
    c = pl.program_id(0)
    kb = pl.program_id(1)

    if True:
        x = x_ref[...]
        e = e_ref[0]
        xe = jax.lax.dot_general(
            x, e, (((1,), (1,)), ((), ())),
            preferred_element_type=jnp.float32,
            precision=jax.lax.Precision.DEFAULT,
        )
        e2row = jax.lax.dot_general(
            jnp.ones((1, D), jnp.float32), e * e, (((1,), (1,)), ((), ())),
            preferred_element_type=jnp.float32,
            precision=jax.lax.Precision.HIGHEST,
        )
        x2col = jax.lax.dot_general(
            x * x, jnp.ones((1, D), jnp.float32), (((1,), (1,)), ((), ())),
            preferred_element_type=jnp.float32,
            precision=jax.lax.Precision.HIGHEST,
        )
        sc_ref[...] = (x2col - 2.0 * xe) + e2row
        scores = sc_ref[...]

        @pl.when(kb == 0)
        def _():
            rm_ref[...] = jnp.full((BATCH, LANES), jnp.inf, jnp.float32)
            rk_ref[...] = jnp.zeros((BATCH, LANES), jnp.int32)

        rm = rm_ref[...]
        rk = rk_ref[...]
        lane_iota = jax.lax.broadcasted_iota(jnp.int32, (BATCH, LANES), 1)
        for s in range(NSLAB):
            slab = scores[:, s * LANES:(s + 1) * LANES]
            kvec = lane_iota + (kb * KCHUNK + s * LANES)
            cond = slab < rm
            rm = jnp.where(cond, slab, rm)
            rk = jnp.where(cond, kvec, rk)
        rm_ref[...] = rm
        rk_ref[...] = rk

        @pl.when(kb == NKB - 1)
        def _():
            lm = jnp.min(rm, axis=1, keepdims=True)
            cand = jnp.where(rm == lm, rk, INT_BIG)
            idx = jnp.min(cand, axis=1, keepdims=True)
            idxsc_ref[0] = idx + c * K
            lane_c = jax.lax.broadcasted_iota(jnp.int32, (BATCH, C), 1)
            aall_ref[...] = jnp.where(lane_c == c, idx, aall_ref[...])

            @pl.when(c == C - 1)
            def _():
                idxbc_ref[...] = aall_ref[...]


def _onehot_kernel(idx2_ref, oh_ref):
    kb = pl.program_id(0)
    idx2 = idx2_ref[...]
    k_iota = (jax.lax.broadcasted_iota(jnp.int32, (BATCH * C, KCHUNK), 1)
              + kb * KCHUNK)
    oh_ref[...] = (k_iota == idx2).astype(jnp.float32)


def _sc_gather(table_flat, idx_flat):
    n_idx = BATCH * C
    half = BATCH // GATHER_WINDOW
    mesh = plsc.VectorSubcoreMesh(core_axis_name="core", subcore_axis_name="subcore")

    @pl.kernel(out_type=jax.ShapeDtypeStruct((BATCH, C * D), jnp.float32), mesh=mesh)
    def gather_kernel(tab_hbm, i_hbm, o_hbm):
        def body(i_vmem, o_vmem):
            pltpu.sync_copy(tab_hbm.at[i_vmem.at[0]], o_vmem)

        pltpu.emit_pipeline(
            body,
            grid=(n_idx // GATHER_WINDOW,),
            in_specs=[pl.BlockSpec((1, GATHER_WINDOW), index_map=lambda i: (0, i))],
            out_specs=[pl.BlockSpec((GATHER_WINDOW, D),
                                    index_map=lambda i: (i % half, i // half))],
            core_axis_name=("core", "subcore"),
            dimension_semantics=(pltpu.PARALLEL,),
        )(i_hbm, o_hbm)

    return gather_kernel(table_flat, idx_flat)


def kernel(cw_q, codebook):
    idx_sc, idx_bc = pl.pallas_call(
        _argmin_kernel,
        grid=(C, NKB),
        in_specs=[
            pl.BlockSpec((BATCH, D), lambda c, kb: (0, c)),
            pl.BlockSpec((1, KCHUNK, D), lambda c, kb: (c, kb, 0)),
        ],
        out_specs=[
            pl.BlockSpec((1, BATCH, 1), lambda c, kb: (c, 0, 0)),
            pl.BlockSpec((BATCH, C), lambda c, kb: (0, 0)),
        ],
        out_shape=[
            jax.ShapeDtypeStruct((C, BATCH, 1), jnp.int32),
            jax.ShapeDtypeStruct((BATCH, C), jnp.int32),
        ],
        scratch_shapes=[
            pltpu.VMEM((BATCH, LANES), jnp.float32),
            pltpu.VMEM((BATCH, LANES), jnp.int32),
            pltpu.VMEM((BATCH, C), jnp.int32),
            pltpu.VMEM((BATCH, KCHUNK), jnp.float32),
        ],
    )(cw_q, codebook)

    idx2 = idx_bc.reshape(BATCH * C, 1)
    one_hot_flat = pl.pallas_call(
        _onehot_kernel,
        grid=(NKB,),
        in_specs=[pl.BlockSpec((BATCH * C, 1), lambda kb: (0, 0))],
        out_specs=pl.BlockSpec((BATCH * C, KCHUNK), lambda kb: (0, kb)),
        out_shape=jax.ShapeDtypeStruct((BATCH * C, K), jnp.float32),
    )(idx2)

    one_hot = one_hot_flat.reshape(BATCH, C, K)
    idx_flat = idx_sc.reshape(1, BATCH * C)
    cw_e = _sc_gather(codebook.reshape(C * K, D), idx_flat)
    cw = cw_q + jax.lax.stop_gradient(cw_e - cw_q)
    return cw, one_hot


# speedup vs baseline: 2.0147x; 1.3312x over previous
import jax
import jax.numpy as jnp
from jax.experimental import pallas as pl
from jax.experimental.pallas import tpu as pltpu
from jax.experimental.pallas import tpu_sc as plsc

BATCH = 256
CW_DIM = 2048
D = 256
K = 8192
C = CW_DIM // D

GATHER_WINDOW = 128
OHCHUNK = 2048
LANES = 128
NSLAB = K // LANES
INT_BIG = 2**31 - 1


def _argmin_kernel(x_ref, e_ref, idxg_ref, idxl_ref):
    c = pl.program_id(0)
    x = x_ref[...]
    e = e_ref[0]
    xe2 = jax.lax.dot_general(
        -2.0 * x, e, (((1,), (1,)), ((), ())),
        preferred_element_type=jnp.float32,
        precision=jax.lax.Precision.DEFAULT,
    )
    e2row = jax.lax.dot_general(
        jnp.ones((1, D), jnp.float32), e * e, (((1,), (1,)), ((), ())),
        preferred_element_type=jnp.float32,
        precision=jax.lax.Precision.HIGHEST,
    )
    x2col = jax.lax.dot_general(
        x * x, jnp.ones((1, D), jnp.float32), (((1,), (1,)), ((), ())),
        preferred_element_type=jnp.float32,
        precision=jax.lax.Precision.HIGHEST,
    )
    scores = (x2col + xe2) + e2row

    rm = jnp.full((BATCH, LANES), jnp.inf, jnp.float32)
    rk = jnp.zeros((BATCH, LANES), jnp.int32)
    lane_iota = jax.lax.broadcasted_iota(jnp.int32, (BATCH, LANES), 1)
    for s in range(NSLAB):
        slab = scores[:, s * LANES:(s + 1) * LANES]
        kvec = lane_iota + s * LANES
        cond = slab < rm
        rm = jnp.where(cond, slab, rm)
        rk = jnp.where(cond, kvec, rk)

    lm = jnp.min(rm, axis=1, keepdims=True)
    cand = jnp.where(rm == lm, rk, INT_BIG)
    idx = jnp.min(cand, axis=1, keepdims=True)
    idxg_ref[0] = idx + c * K
    idxl_ref[0] = idx


def _onehot_kernel(idx2_ref, oh_ref):
    kb = pl.program_id(0)
    idx2 = idx2_ref[...]
    k_iota = (jax.lax.broadcasted_iota(jnp.int32, (BATCH * C, OHCHUNK), 1)
              + kb * OHCHUNK)
    oh_ref[...] = (k_iota == idx2).astype(jnp.float32)


def _sc_gather(table_flat, idx_flat):
    n_idx = BATCH * C
    half = BATCH // GATHER_WINDOW
    mesh = plsc.VectorSubcoreMesh(core_axis_name="core", subcore_axis_name="subcore")

    @pl.kernel(out_type=jax.ShapeDtypeStruct((BATCH, C * D), jnp.float32), mesh=mesh)
    def gather_kernel(tab_hbm, i_hbm, o_hbm):
        def body(i_vmem, o_vmem):
            pltpu.sync_copy(tab_hbm.at[i_vmem.at[0]], o_vmem)

        pltpu.emit_pipeline(
            body,
            grid=(n_idx // GATHER_WINDOW,),
            in_specs=[pl.BlockSpec((1, GATHER_WINDOW), index_map=lambda i: (0, i))],
            out_specs=[pl.BlockSpec((GATHER_WINDOW, D),
                                    index_map=lambda i: (i % half, i // half))],
            core_axis_name=("core", "subcore"),
            dimension_semantics=(pltpu.PARALLEL,),
        )(i_hbm, o_hbm)

    return gather_kernel(table_flat, idx_flat)


def kernel(cw_q, codebook):
    idx_g, idx_l = pl.pallas_call(
        _argmin_kernel,
        grid=(C,),
        in_specs=[
            pl.BlockSpec((BATCH, D), lambda c: (0, c)),
            pl.BlockSpec((1, K, D), lambda c: (c, 0, 0)),
        ],
        out_specs=[
            pl.BlockSpec((1, BATCH, 1), lambda c: (c, 0, 0)),
            pl.BlockSpec((1, BATCH, 1), lambda c: (c, 0, 0)),
        ],
        out_shape=[
            jax.ShapeDtypeStruct((C, BATCH, 1), jnp.int32),
            jax.ShapeDtypeStruct((C, BATCH, 1), jnp.int32),
        ],
        compiler_params=pltpu.CompilerParams(
            dimension_semantics=("parallel",)),
    )(cw_q, codebook)

    idx2 = idx_l[:, :, 0].T.reshape(BATCH * C, 1)
    one_hot_flat = pl.pallas_call(
        _onehot_kernel,
        grid=(K // OHCHUNK,),
        in_specs=[pl.BlockSpec((BATCH * C, 1), lambda kb: (0, 0))],
        out_specs=pl.BlockSpec((BATCH * C, OHCHUNK), lambda kb: (0, kb)),
        out_shape=jax.ShapeDtypeStruct((BATCH * C, K), jnp.float32),
        compiler_params=pltpu.CompilerParams(
            dimension_semantics=("parallel",)),
    )(idx2)

    one_hot = one_hot_flat.reshape(BATCH, C, K)
    idx_flat = idx_g.reshape(1, BATCH * C)
    cw_e = _sc_gather(codebook.reshape(C * K, D), idx_flat)
    cw = cw_q + jax.lax.stop_gradient(cw_e - cw_q)
    return cw, one_hot
